# Initial kernel scaffold; baseline (speedup 1.0000x reference)
import jax
import jax.numpy as jnp
from jax.experimental import pallas as pl


def kernel(values_a, metadata_a, values_b, metadata_b, Wq, Wk, Wv, Wo):
    raise NotImplementedError("write your pallas kernel here")


# speedup vs baseline: 1.4740x; 1.4740x over previous
import functools

import jax
import jax.numpy as jnp
import numpy as np
from jax.experimental import pallas as pl


def _segments(n, k):
    stride = n // k
    idx = np.linspace(0, n - 1, k).astype(np.int64)
    d = idx - stride * np.arange(k)
    segs = []
    s0 = 0
    for i in range(1, k + 1):
        if i == k or d[i] != d[s0]:
            segs.append((int(s0), int(i), int(d[s0])))
            s0 = i
    return stride, segs


def _gather_kernel(segs, vra, mra, vrb, mrb, xa, xb):
    g = pl.program_id(1)
    for s0, s1, d in segs:
        @pl.when(g == d)
        def _(s0=s0, s1=s1):
            vd = vra.shape[2]
            md = mra.shape[2]
            xa[0, s0:s1, 0:vd] = vra[0, s0:s1, :]
            xa[0, s0:s1, vd:vd + md] = mra[0, s0:s1, :]
            xb[0, s0:s1, 0:vd] = vrb[0, s0:s1, :]
            xb[0, s0:s1, vd:vd + md] = mrb[0, s0:s1, :]


def _qkv_kernel(vd, x, wq, wk, wv, q, k, v):
    xx = x[0]
    q[0] = jnp.dot(xx, wq[...], preferred_element_type=jnp.float32)
    k[0] = jnp.dot(xx, wk[...], preferred_element_type=jnp.float32)
    v[0] = jnp.dot(xx[:, :vd], wv[...], preferred_element_type=jnp.float32)


def _attn_kernel(scale, dh, q, k, v, wo, u):
    h = pl.program_id(1)
    qq, kk, vv, woo = q[0], k[0], v[0], wo[...]
    contrib = None
    for j in range(qq.shape[-1] // dh):
        qh = qq[:, j * dh:(j + 1) * dh]
        kh = kk[:, j * dh:(j + 1) * dh]
        vh = vv[:, j * dh:(j + 1) * dh]
        s = jax.lax.dot_general(qh, kh, (((1,), (1,)), ((), ())),
                                preferred_element_type=jnp.float32) * scale
        m = jnp.max(s, axis=-1, keepdims=True)
        p = jnp.exp(s - m)
        l = jnp.sum(p, axis=-1, keepdims=True)
        o = jnp.dot(p, vh, preferred_element_type=jnp.float32) / l
        c = jnp.dot(o, woo[j * dh:(j + 1) * dh, :],
                    preferred_element_type=jnp.float32)
        contrib = c if contrib is None else contrib + c

    @pl.when(h == 0)
    def _():
        u[0] = contrib

    @pl.when(h > 0)
    def _():
        u[0] += contrib


def _scatter_kernel(segs, vr, uu, o):
    g = pl.program_id(1)
    o[0] = vr[0]
    for s0, s1, d in segs:
        @pl.when(g == d)
        def _(s0=s0, s1=s1):
            o[0, s0:s1, :] += uu[0, 0, s0:s1, :]


def kernel(values_a, metadata_a, values_b, metadata_b, Wq, Wk, Wv, Wo):
    B, N, VD = values_a.shape
    MD = metadata_a.shape[2]
    ID = Wq.shape[1]
    K = ID
    H = 16
    dh = ID // H

    stride, segs = _segments(N, K)
    R = N // stride

    vra = values_a.reshape(B, R, stride * VD)
    vrb = values_b.reshape(B, R, stride * VD)
    mra = metadata_a.reshape(B, R, stride * MD)
    mrb = metadata_b.reshape(B, R, stride * MD)

    xspec = pl.BlockSpec((1, K, VD + MD), lambda b, g: (b, 0, 0))
    xa, xb = pl.pallas_call(
        functools.partial(_gather_kernel, segs),
        grid=(B, stride),
        in_specs=[
            pl.BlockSpec((1, R, VD), lambda b, g: (b, 0, g)),
            pl.BlockSpec((1, R, MD), lambda b, g: (b, 0, g)),
            pl.BlockSpec((1, R, VD), lambda b, g: (b, 0, g)),
            pl.BlockSpec((1, R, MD), lambda b, g: (b, 0, g)),
        ],
        out_specs=[xspec, xspec],
        out_shape=[jax.ShapeDtypeStruct((B, K, VD + MD), jnp.float32)] * 2,
    )(vra, mra, vrb, mrb)

    x = jnp.concatenate([xa, xb], axis=1)
    T = 2 * K

    RB = 2
    q, k, v = pl.pallas_call(
        functools.partial(_qkv_kernel, VD),
        grid=(B, RB),
        in_specs=[
            pl.BlockSpec((1, T // RB, VD + MD), lambda b, r: (b, r, 0)),
            pl.BlockSpec((VD + MD, ID), lambda b, r: (0, 0)),
            pl.BlockSpec((VD + MD, ID), lambda b, r: (0, 0)),
            pl.BlockSpec((VD, ID), lambda b, r: (0, 0)),
        ],
        out_specs=[pl.BlockSpec((1, T // RB, ID), lambda b, r: (b, r, 0))] * 3,
        out_shape=[jax.ShapeDtypeStruct((B, T, ID), jnp.float32)] * 3,
    )(x, Wq, Wk, Wv)

    HPB = 2
    hspec = pl.BlockSpec((1, T, HPB * dh), lambda b, h: (b, 0, h))
    u = pl.pallas_call(
        functools.partial(_attn_kernel, 1.0 / np.sqrt(dh), dh),
        grid=(B, H // HPB),
        in_specs=[
            hspec, hspec, hspec,
            pl.BlockSpec((HPB * dh, VD), lambda b, h: (h, 0)),
        ],
        out_specs=pl.BlockSpec((1, T, VD), lambda b, h: (b, 0, 0)),
        out_shape=jax.ShapeDtypeStruct((B, T, VD), jnp.float32),
    )(q, k, v, Wo)

    ur = u.reshape(B, 2, K, VD)

    outs = []
    for src, vr in ((0, vra), (1, vrb)):
        o = pl.pallas_call(
            functools.partial(_scatter_kernel, segs),
            grid=(B, stride),
            in_specs=[
                pl.BlockSpec((1, R, VD), lambda b, g: (b, 0, g)),
                pl.BlockSpec((1, 1, K, VD), lambda b, g, src=src: (b, src, 0, 0)),
            ],
            out_specs=pl.BlockSpec((1, R, VD), lambda b, g: (b, 0, g)),
            out_shape=jax.ShapeDtypeStruct((B, R, stride * VD), jnp.float32),
        )(vr, ur)
        outs.append(o.reshape(B, N, VD))
    return outs[0], outs[1]


# speedup vs baseline: 1.5042x; 1.0205x over previous
import functools

import jax
import jax.numpy as jnp
import numpy as np
from jax.experimental import pallas as pl


def _segments(n, k):
    stride = n // k
    idx = np.linspace(0, n - 1, k).astype(np.int64)
    d = idx - stride * np.arange(k)
    segs = []
    s0 = 0
    for i in range(1, k + 1):
        if i == k or d[i] != d[s0]:
            segs.append((int(s0), int(i), int(d[s0])))
            s0 = i
    return stride, segs


def _gather_kernel(segs, kk, vra, mra, vrb, mrb, x):
    g = pl.program_id(1)
    for s0, s1, d in segs:
        @pl.when(g == d)
        def _(s0=s0, s1=s1):
            vd = vra.shape[2]
            md = mra.shape[2]
            x[0, s0:s1, 0:vd] = vra[0, s0:s1, :]
            x[0, s0:s1, vd:vd + md] = mra[0, s0:s1, :]
            x[0, kk + s0:kk + s1, 0:vd] = vrb[0, s0:s1, :]
            x[0, kk + s0:kk + s1, vd:vd + md] = mrb[0, s0:s1, :]


def _qkv_kernel(vd, x, wq, wk, wv, q, k, v):
    xx = x[0].astype(jnp.bfloat16)
    q[0] = jnp.dot(xx, wq[...].astype(jnp.bfloat16),
                   preferred_element_type=jnp.float32)
    k[0] = jnp.dot(xx, wk[...].astype(jnp.bfloat16),
                   preferred_element_type=jnp.float32)
    v[0] = jnp.dot(xx[:, :vd], wv[...].astype(jnp.bfloat16),
                   preferred_element_type=jnp.float32)


def _attn_kernel(scale, dh, q, k, v, wo, u):
    h = pl.program_id(1)
    qq = q[0].astype(jnp.bfloat16)
    kk = k[0].astype(jnp.bfloat16)
    vv = v[0].astype(jnp.bfloat16)
    woo = wo[...].astype(jnp.bfloat16)
    contrib = None
    for j in range(qq.shape[-1] // dh):
        qh = qq[:, j * dh:(j + 1) * dh]
        kh = kk[:, j * dh:(j + 1) * dh]
        vh = vv[:, j * dh:(j + 1) * dh]
        s = jax.lax.dot_general(qh, kh, (((1,), (1,)), ((), ())),
                                preferred_element_type=jnp.float32) * scale
        m = jnp.max(s, axis=-1, keepdims=True)
        p = jnp.exp(s - m)
        l = jnp.sum(p, axis=-1, keepdims=True)
        o = jnp.dot(p.astype(jnp.bfloat16), vh,
                    preferred_element_type=jnp.float32) / l
        c = jnp.dot(o.astype(jnp.bfloat16), woo[j * dh:(j + 1) * dh, :],
                    preferred_element_type=jnp.float32)
        contrib = c if contrib is None else contrib + c

    @pl.when(h == 0)
    def _():
        u[0] = contrib

    @pl.when(h > 0)
    def _():
        u[0] += contrib


def _scatter_kernel(segs, vr, uu, o):
    g = pl.program_id(1)
    o[0] = vr[0]
    for s0, s1, d in segs:
        @pl.when(g == d)
        def _(s0=s0, s1=s1):
            o[0, s0:s1, :] += uu[0, 0, s0:s1, :]


def kernel(values_a, metadata_a, values_b, metadata_b, Wq, Wk, Wv, Wo):
    B, N, VD = values_a.shape
    MD = metadata_a.shape[2]
    ID = Wq.shape[1]
    K = ID
    H = 16
    dh = ID // H

    stride, segs = _segments(N, K)
    R = N // stride

    vra = values_a.reshape(B, R, stride * VD)
    vrb = values_b.reshape(B, R, stride * VD)
    mra = metadata_a.reshape(B, R, stride * MD)
    mrb = metadata_b.reshape(B, R, stride * MD)

    T = 2 * K
    x = pl.pallas_call(
        functools.partial(_gather_kernel, segs, K),
        grid=(B, stride),
        in_specs=[
            pl.BlockSpec((1, R, VD), lambda b, g: (b, 0, g)),
            pl.BlockSpec((1, R, MD), lambda b, g: (b, 0, g)),
            pl.BlockSpec((1, R, VD), lambda b, g: (b, 0, g)),
            pl.BlockSpec((1, R, MD), lambda b, g: (b, 0, g)),
        ],
        out_specs=pl.BlockSpec((1, T, VD + MD), lambda b, g: (b, 0, 0)),
        out_shape=jax.ShapeDtypeStruct((B, T, VD + MD), jnp.float32),
    )(vra, mra, vrb, mrb)

    RB = 2
    q, k, v = pl.pallas_call(
        functools.partial(_qkv_kernel, VD),
        grid=(B, RB),
        in_specs=[
            pl.BlockSpec((1, T // RB, VD + MD), lambda b, r: (b, r, 0)),
            pl.BlockSpec((VD + MD, ID), lambda b, r: (0, 0)),
            pl.BlockSpec((VD + MD, ID), lambda b, r: (0, 0)),
            pl.BlockSpec((VD, ID), lambda b, r: (0, 0)),
        ],
        out_specs=[pl.BlockSpec((1, T // RB, ID), lambda b, r: (b, r, 0))] * 3,
        out_shape=[jax.ShapeDtypeStruct((B, T, ID), jnp.float32)] * 3,
    )(x, Wq, Wk, Wv)

    HPB = 2
    hspec = pl.BlockSpec((1, T, HPB * dh), lambda b, h: (b, 0, h))
    u = pl.pallas_call(
        functools.partial(_attn_kernel, 1.0 / np.sqrt(dh), dh),
        grid=(B, H // HPB),
        in_specs=[
            hspec, hspec, hspec,
            pl.BlockSpec((HPB * dh, VD), lambda b, h: (h, 0)),
        ],
        out_specs=pl.BlockSpec((1, T, VD), lambda b, h: (b, 0, 0)),
        out_shape=jax.ShapeDtypeStruct((B, T, VD), jnp.float32),
    )(q, k, v, Wo)

    ur = u.reshape(B, 2, K, VD)

    outs = []
    for src, vr in ((0, vra), (1, vrb)):
        o = pl.pallas_call(
            functools.partial(_scatter_kernel, segs),
            grid=(B, stride),
            in_specs=[
                pl.BlockSpec((1, R, VD), lambda b, g: (b, 0, g)),
                pl.BlockSpec((1, 1, K, VD), lambda b, g, src=src: (b, src, 0, 0)),
            ],
            out_specs=pl.BlockSpec((1, R, VD), lambda b, g: (b, 0, g)),
            out_shape=jax.ShapeDtypeStruct((B, R, stride * VD), jnp.float32),
        )(vr, ur)
        outs.append(o.reshape(B, N, VD))
    return outs[0], outs[1]
